# Initial kernel scaffold; baseline (speedup 1.0000x reference)
import jax
import jax.numpy as jnp
from jax.experimental import pallas as pl


def kernel(image_sizes, image_sizes_ori, classifications, regressions, anchors, score_thresh, nms_thresh):
    raise NotImplementedError("write your pallas kernel here")


# speedup vs baseline: 153.9864x; 153.9864x over previous
import jax
import jax.numpy as jnp
from jax.experimental import pallas as pl
from jax.experimental.pallas import tpu as pltpu

_NP = 5120
_TQ = 128


def _scores_kernel(cls_ref, s_ref, l_ref):
    x = cls_ref[0]
    s_ref[0] = jnp.max(x, axis=-1, keepdims=True)
    l_ref[0] = jnp.argmax(x, axis=-1, keepdims=True).astype(jnp.int32)


def _decode_rows(reg, anch, wf, hf):
    al, at, ar, ab = anch[0:1], anch[1:2], anch[2:3], anch[3:4]
    xr, yr, wr, hr = reg[0:1], reg[1:2], reg[2:3], reg[3:4]
    return _decode(al, at, ar, ab, xr, yr, wr, hr, wf, hf)


def _decode_cols(reg, anch, wf, hf):
    al, at, ar, ab = anch[:, 0:1], anch[:, 1:2], anch[:, 2:3], anch[:, 3:4]
    xr, yr, wr, hr = reg[:, 0:1], reg[:, 1:2], reg[:, 2:3], reg[:, 3:4]
    return _decode(al, at, ar, ab, xr, yr, wr, hr, wf, hf)


def _decode(al, at, ar, ab, xr, yr, wr, hr, wf, hf):
    wa = ar - al
    ha = ab - at
    cxa = al + wa / 2
    cya = at + ha / 2
    cx = cxa + xr * wa
    cy = cya + yr * ha
    wt = wa * jnp.exp(wr)
    ht = ha * jnp.exp(hr)
    l = cx - wt / 2
    t = cy - ht / 2
    r = l + wt
    b = t + ht
    l = jnp.clip(l, 0.0, wf - 1.0)
    r = jnp.clip(r, 0.0, wf - 1.0)
    t = jnp.clip(t, 0.0, hf - 1.0)
    b = jnp.clip(b, 0.0, hf - 1.0)
    return l, t, r, b


def _nms_kernel(sizes_ref, orisz_ref, thr_ref,
                s_row_ref, l_row_ref, reg_row_ref, anch_row_ref,
                s_col_ref, l_col_ref, reg_col_ref, anch_col_ref,
                boxes_ref, sco_ref, lab_ref, keep_ref):
    i = pl.program_id(0)
    hf = sizes_ref[i, 0].astype(jnp.float32)
    wf = sizes_ref[i, 1].astype(jnp.float32)
    sth = thr_ref[0]
    nth = thr_ref[1]

    l, t, r, b = _decode_rows(reg_row_ref[0], anch_row_ref[0], wf, hf)
    m1 = jnp.maximum(jnp.maximum(jnp.max(l), jnp.max(t)),
                     jnp.maximum(jnp.max(r), jnp.max(b))) + 1.0
    s_row = s_row_ref[0]
    lab_row = l_row_ref[0]
    off = lab_row.astype(jnp.float32) * m1
    ol, ot, orr, ob = l + off, t + off, r + off, b + off
    area_r = (jnp.clip(orr - ol, 0.0, None) * jnp.clip(ob - ot, 0.0, None))
    valid_row = s_row >= sth
    idx_row = jax.lax.broadcasted_iota(jnp.int32, (1, _NP), 1)

    lc, tc, rc, bc = _decode_cols(reg_col_ref[0], anch_col_ref[0], wf, hf)
    s_col = s_col_ref[0]
    offc = l_col_ref[0].astype(jnp.float32) * m1
    olc, otc, orc, obc = lc + offc, tc + offc, rc + offc, bc + offc
    area_c = (jnp.clip(orc - olc, 0.0, None) * jnp.clip(obc - otc, 0.0, None))
    valid_colf = (s_col >= sth).astype(jnp.float32)
    idx_col = jax.lax.broadcasted_iota(jnp.int32, (_NP, 1), 0)

    validf = valid_row.astype(jnp.bfloat16)

    def one_round(keep):
        supp = jnp.zeros((1, _NP), jnp.float32)
        for q in range(_NP // _TQ):
            sl = slice(q * _TQ, (q + 1) * _TQ)
            jl, jt, jr, jb = olc[sl], otc[sl], orc[sl], obc[sl]
            ja, js, jv, ji = area_c[sl], s_col[sl], valid_colf[sl], idx_col[sl]
            ltx = jnp.maximum(jl, ol)
            lty = jnp.maximum(jt, ot)
            rbx = jnp.minimum(jr, orr)
            rby = jnp.minimum(jb, ob)
            wx = jnp.clip(rbx - ltx, 0.0, None)
            wy = jnp.clip(rby - lty, 0.0, None)
            inter = wx * wy
            union = ja + area_r - inter
            iou = inter / jnp.maximum(union, 1e-9)
            prec = (js > s_row) | ((js == s_row) & (ji < idx_row))
            ablk = ((iou > nth) & prec & (jv > 0.5)).astype(jnp.bfloat16)
            kq = keep[:, sl]
            supp = supp + jax.lax.dot_general(
                kq, ablk, (((1,), (0,)), ((), ())),
                preferred_element_type=jnp.float32)
        return supp

    def cond(c):
        return c[1]

    def body(c):
        keep, _ = c
        supp = one_round(keep)
        knew = jnp.where(supp < 0.5, validf, jnp.zeros_like(validf))
        delta = jnp.max(jnp.abs(knew.astype(jnp.float32) -
                                keep.astype(jnp.float32)))
        return knew, delta > 0.5

    keep, _ = jax.lax.while_loop(cond, body, (validf, jnp.bool_(True)))
    keepb = keep > jnp.bfloat16(0.5)

    scale = orisz_ref[i, 0].astype(jnp.float32) / sizes_ref[i, 0].astype(jnp.float32)
    zero = jnp.zeros_like(l)
    boxes_ref[0] = jnp.concatenate(
        [jnp.where(keepb, l * scale, zero),
         jnp.where(keepb, t * scale, zero),
         jnp.where(keepb, r * scale, zero),
         jnp.where(keepb, b * scale, zero)], axis=0)
    sco_ref[0] = jnp.where(keepb, s_row, zero)
    lab_ref[0] = jnp.where(keepb, lab_row, -1)
    keep_ref[0] = keepb.astype(jnp.int32)


def kernel(image_sizes, image_sizes_ori, classifications, regressions,
           anchors, score_thresh, nms_thresh):
    B, n, c = classifications.shape
    s_col, l_col = pl.pallas_call(
        _scores_kernel,
        grid=(B,),
        in_specs=[pl.BlockSpec((1, n, c), lambda i: (i, 0, 0))],
        out_specs=[pl.BlockSpec((1, n, 1), lambda i: (i, 0, 0)),
                   pl.BlockSpec((1, n, 1), lambda i: (i, 0, 0))],
        out_shape=[jax.ShapeDtypeStruct((B, n, 1), jnp.float32),
                   jax.ShapeDtypeStruct((B, n, 1), jnp.int32)],
        compiler_params=pltpu.CompilerParams(dimension_semantics=("parallel",)),
    )(classifications)

    pad = _NP - n
    s_colp = jnp.pad(s_col, ((0, 0), (0, pad), (0, 0)),
                     constant_values=-jnp.inf)
    l_colp = jnp.pad(l_col, ((0, 0), (0, pad), (0, 0)))
    s_rowp = s_colp.reshape(B, 1, _NP)
    l_rowp = l_colp.reshape(B, 1, _NP)
    reg_colp = jnp.pad(regressions, ((0, 0), (0, pad), (0, 0)))
    reg_rowp = jnp.transpose(reg_colp, (0, 2, 1))
    anch_colp = jnp.pad(anchors, ((0, pad), (0, 0)))[None]
    anch_rowp = jnp.transpose(anch_colp, (0, 2, 1))
    thr = jnp.stack([jnp.asarray(score_thresh, jnp.float32),
                     jnp.asarray(nms_thresh, jnp.float32)])

    boxes4, sco, lab, keepi = pl.pallas_call(
        _nms_kernel,
        grid=(B,),
        in_specs=[
            pl.BlockSpec(memory_space=pltpu.SMEM),
            pl.BlockSpec(memory_space=pltpu.SMEM),
            pl.BlockSpec(memory_space=pltpu.SMEM),
            pl.BlockSpec((1, 1, _NP), lambda i: (i, 0, 0)),
            pl.BlockSpec((1, 1, _NP), lambda i: (i, 0, 0)),
            pl.BlockSpec((1, 4, _NP), lambda i: (i, 0, 0)),
            pl.BlockSpec((1, 4, _NP), lambda i: (0, 0, 0)),
            pl.BlockSpec((1, _NP, 1), lambda i: (i, 0, 0)),
            pl.BlockSpec((1, _NP, 1), lambda i: (i, 0, 0)),
            pl.BlockSpec((1, _NP, 4), lambda i: (i, 0, 0)),
            pl.BlockSpec((1, _NP, 4), lambda i: (0, 0, 0)),
        ],
        out_specs=[pl.BlockSpec((1, 4, _NP), lambda i: (i, 0, 0)),
                   pl.BlockSpec((1, 1, _NP), lambda i: (i, 0, 0)),
                   pl.BlockSpec((1, 1, _NP), lambda i: (i, 0, 0)),
                   pl.BlockSpec((1, 1, _NP), lambda i: (i, 0, 0))],
        out_shape=[jax.ShapeDtypeStruct((B, 4, _NP), jnp.float32),
                   jax.ShapeDtypeStruct((B, 1, _NP), jnp.float32),
                   jax.ShapeDtypeStruct((B, 1, _NP), jnp.int32),
                   jax.ShapeDtypeStruct((B, 1, _NP), jnp.int32)],
        compiler_params=pltpu.CompilerParams(dimension_semantics=("parallel",)),
    )(image_sizes, image_sizes_ori, thr, s_rowp, l_rowp, reg_rowp, anch_rowp,
      s_colp, l_colp, reg_colp, anch_colp)

    boxes = jnp.transpose(boxes4, (0, 2, 1))[:, :n, :]
    scores = sco[:, 0, :n]
    labels = lab[:, 0, :n]
    keep = keepi[:, 0, :n].astype(bool)
    return boxes, scores, labels, keep


# speedup vs baseline: 436.3389x; 2.8336x over previous
import dataclasses
import functools

import jax
import jax.numpy as jnp
from jax import lax
from jax.experimental import pallas as pl
from jax.experimental.pallas import tpu as pltpu
from jax.experimental.pallas import tpu_sc as plsc

_NP = 5120
_L = 16
_NCHB = _NP // _L
_CAP = 5024
_NEG = -1e30


def _scores_kernel(cls_ref, s_ref, l_ref):
    x = cls_ref[0]
    s_ref[0] = jnp.max(x, axis=-1, keepdims=True)
    l_ref[0] = jnp.argmax(x, axis=-1, keepdims=True).astype(jnp.int32)


def _decode_rows(reg, anch, wf, hf):
    al, at, ar, ab = anch[0:1], anch[1:2], anch[2:3], anch[3:4]
    xr, yr, wr, hr = reg[0:1], reg[1:2], reg[2:3], reg[3:4]
    wa = ar - al
    ha = ab - at
    cxa = al + wa / 2
    cya = at + ha / 2
    cx = cxa + xr * wa
    cy = cya + yr * ha
    wt = wa * jnp.exp(wr)
    ht = ha * jnp.exp(hr)
    l = cx - wt / 2
    t = cy - ht / 2
    r = l + wt
    b = t + ht
    l = jnp.clip(l, 0.0, wf - 1.0)
    r = jnp.clip(r, 0.0, wf - 1.0)
    t = jnp.clip(t, 0.0, hf - 1.0)
    b = jnp.clip(b, 0.0, hf - 1.0)
    return l, t, r, b


def _prep_kernel(sizes_ref, thr_ref, s_row_ref, l_row_ref, reg_row_ref,
                 anch_row_ref, data_ref):
    i = pl.program_id(0)
    hf = sizes_ref[i, 0].astype(jnp.float32)
    wf = sizes_ref[i, 1].astype(jnp.float32)
    l, t, r, b = _decode_rows(reg_row_ref[0], anch_row_ref[0], wf, hf)
    m1 = jnp.maximum(jnp.maximum(jnp.max(l), jnp.max(t)),
                     jnp.maximum(jnp.max(r), jnp.max(b))) + 1.0
    s_row = s_row_ref[0]
    labf = l_row_ref[0].astype(jnp.float32)
    off = labf * m1
    valid = s_row >= thr_ref[0]
    labm = jnp.where(valid, labf, jnp.full_like(labf, -1.0))
    data_ref[0] = jnp.concatenate(
        [l + off, t + off, r + off, b + off, s_row, labm], axis=0)


def _sc_nms_body(data_hbm, thr_hbm, keep_hbm,
                 st_ol, st_ot, st_or, st_ob, st_sc, st_lab,
                 olm, otm, orm, obm, scm, idxm, alivem,
                 keepfull, tmpm, thrv, spmem):
    core = lax.axis_index("c")
    s = lax.axis_index("s")
    lb = s // 8
    off_c = s % 8
    b = core * 2 + lb

    pltpu.sync_copy(thr_hbm, thrv)
    pltpu.sync_copy(data_hbm.at[b, 0], st_ol)
    pltpu.sync_copy(data_hbm.at[b, 1], st_ot)
    pltpu.sync_copy(data_hbm.at[b, 2], st_or)
    pltpu.sync_copy(data_hbm.at[b, 3], st_ob)
    pltpu.sync_copy(data_hbm.at[b, 4], st_sc)
    pltpu.sync_copy(data_hbm.at[b, 5], st_lab)
    nthv = thrv[...]

    zero16i = jnp.zeros((_L,), jnp.int32)

    def zf(k, _):
        keepfull[pl.ds(k * _L, _L)] = zero16i
        return 0

    lax.fori_loop(0, _NCHB, zf, 0)

    iota = lax.iota(jnp.int32, _L)

    @pl.loop(0, 10)
    def _class_loop(kk):
        c = off_c + 8 * kk
        cvec = jnp.full((_L,), c.astype(jnp.float32))

        def comp(k, cnt):
            sl = pl.ds(k * _L, _L)
            msk = st_lab[sl] == cvec
            cum = plsc.cumsum(msk.astype(jnp.int32))
            pos = jnp.where(msk, jnp.full((_L,), cnt) + cum - 1,
                            jnp.full((_L,), _CAP - 1))
            plsc.store_scatter(olm, [pos], st_ol[sl])
            plsc.store_scatter(otm, [pos], st_ot[sl])
            plsc.store_scatter(orm, [pos], st_or[sl])
            plsc.store_scatter(obm, [pos], st_ob[sl])
            plsc.store_scatter(scm, [pos], st_sc[sl])
            plsc.store_scatter(idxm, [pos], iota + k * _L)
            dc = jnp.max(plsc.all_reduce_population_count(msk))
            return cnt + dc

        m = lax.fori_loop(0, _NCHB, comp, jnp.int32(0))
        nch = (m + _L - 1) // _L
        mv = jnp.full((_L,), m)

        def ainit(k, _):
            alivem[pl.ds(k * _L, _L)] = jnp.where(
                iota + k * _L < mv, jnp.full((_L,), 1.0), jnp.full((_L,), 0.0))
            return 0

        lax.fori_loop(0, nch, ainit, 0)

        def first_scan(k, carry):
            best, bch = carry
            sl = pl.ds(k * _L, _L)
            cm = jnp.max(jnp.where(alivem[sl] > 0.5, scm[sl],
                                   jnp.full((_L,), _NEG)))
            take = cm > best
            return (jnp.where(take, cm, best), jnp.where(take, k, bch))

        best0 = lax.fori_loop(0, nch, first_scan,
                              (jnp.float32(_NEG), jnp.int32(0)))

        def wcond(st):
            return st[0] > jnp.float32(-1e29)

        def wbody(st):
            best, bch = st
            sl = pl.ds(bch * _L, _L)
            msk = (scm[sl] == jnp.full((_L,), best)) & (alivem[sl] > 0.5)
            lane = jnp.max(plsc.all_reduce_ffs(msk))
            widx = bch * _L + lane
            wvec = jnp.full((_L,), widx)
            worig = jnp.max(plsc.load_gather(idxm, [wvec]))
            plsc.store_scatter(keepfull, [jnp.full((_L,), worig)],
                               jnp.full((_L,), 1, jnp.int32))
            wl = plsc.load_gather(olm, [wvec])
            wt = plsc.load_gather(otm, [wvec])
            wr = plsc.load_gather(orm, [wvec])
            wb = plsc.load_gather(obm, [wvec])
            wa = (jnp.maximum(wr - wl, 0.0) * jnp.maximum(wb - wt, 0.0))

            def sscan(k, carry):
                nb, nbc = carry
                sl2 = pl.ds(k * _L, _L)
                li, ti = olm[sl2], otm[sl2]
                ri, bi = orm[sl2], obm[sl2]
                ltx = jnp.maximum(wl, li)
                lty = jnp.maximum(wt, ti)
                rbx = jnp.minimum(wr, ri)
                rby = jnp.minimum(wb, bi)
                wx = jnp.maximum(rbx - ltx, 0.0)
                wy = jnp.maximum(rby - lty, 0.0)
                inter = wx * wy
                ai = (jnp.maximum(ri - li, 0.0) * jnp.maximum(bi - ti, 0.0))
                union = (ai + wa) - inter
                iou = inter / jnp.maximum(union, 1e-9)
                newal = jnp.where(iou > nthv, jnp.full((_L,), 0.0),
                                  alivem[sl2])
                alivem[sl2] = newal
                cm = jnp.max(jnp.where(newal > 0.5, scm[sl2],
                                       jnp.full((_L,), _NEG)))
                take = cm > nb
                return (jnp.where(take, cm, nb), jnp.where(take, k, nbc))

            return lax.fori_loop(0, nch, sscan,
                                 (jnp.float32(_NEG), jnp.int32(0)))

        lax.while_loop(wcond, wbody, best0)

    pltpu.sync_copy(keepfull, spmem.at[s])
    plsc.subcore_barrier()

    @pl.when(off_c == 0)
    def _merge():
        for j in range(1, 8):
            pltpu.sync_copy(spmem.at[s + j], tmpm)

            def om(k, _):
                sl = pl.ds(k * _L, _L)
                keepfull[sl] = keepfull[sl] | tmpm[sl]
                return 0

            lax.fori_loop(0, _NCHB, om, 0)
        pltpu.sync_copy(keepfull, keep_hbm.at[b])


def _final_kernel(sizes_ref, orisz_ref, s_row_ref, l_row_ref, reg_row_ref,
                  anch_row_ref, keep_row_ref,
                  boxes_ref, sco_ref, lab_ref, keep_ref):
    i = pl.program_id(0)
    hf = sizes_ref[i, 0].astype(jnp.float32)
    wf = sizes_ref[i, 1].astype(jnp.float32)
    l, t, r, b = _decode_rows(reg_row_ref[0], anch_row_ref[0], wf, hf)
    s_row = s_row_ref[0]
    lab_row = l_row_ref[0]
    keepb = keep_row_ref[0] > 0
    scale = orisz_ref[i, 0].astype(jnp.float32) / sizes_ref[i, 0].astype(jnp.float32)
    zero = jnp.zeros_like(l)
    boxes_ref[0] = jnp.concatenate(
        [jnp.where(keepb, l * scale, zero),
         jnp.where(keepb, t * scale, zero),
         jnp.where(keepb, r * scale, zero),
         jnp.where(keepb, b * scale, zero)], axis=0)
    sco_ref[0] = jnp.where(keepb, s_row, zero)
    lab_ref[0] = jnp.where(keepb, lab_row, -1)
    keep_ref[0] = keepb.astype(jnp.int32)


def kernel(image_sizes, image_sizes_ori, classifications, regressions,
           anchors, score_thresh, nms_thresh):
    B, n, c = classifications.shape
    s_col, l_col = pl.pallas_call(
        _scores_kernel,
        grid=(B,),
        in_specs=[pl.BlockSpec((1, n, c), lambda i: (i, 0, 0))],
        out_specs=[pl.BlockSpec((1, n, 1), lambda i: (i, 0, 0)),
                   pl.BlockSpec((1, n, 1), lambda i: (i, 0, 0))],
        out_shape=[jax.ShapeDtypeStruct((B, n, 1), jnp.float32),
                   jax.ShapeDtypeStruct((B, n, 1), jnp.int32)],
        compiler_params=pltpu.CompilerParams(dimension_semantics=("parallel",)),
    )(classifications)

    pad = _NP - n
    s_rowp = jnp.pad(s_col, ((0, 0), (0, pad), (0, 0)),
                     constant_values=-jnp.inf).reshape(B, 1, _NP)
    l_rowp = jnp.pad(l_col, ((0, 0), (0, pad), (0, 0))).reshape(B, 1, _NP)
    reg_rowp = jnp.transpose(jnp.pad(regressions, ((0, 0), (0, pad), (0, 0))),
                             (0, 2, 1))
    anch_rowp = jnp.transpose(jnp.pad(anchors, ((0, pad), (0, 0))), (1, 0))[None]
    thr = jnp.stack([jnp.asarray(score_thresh, jnp.float32),
                     jnp.asarray(nms_thresh, jnp.float32)])

    data = pl.pallas_call(
        _prep_kernel,
        grid=(B,),
        in_specs=[
            pl.BlockSpec(memory_space=pltpu.SMEM),
            pl.BlockSpec(memory_space=pltpu.SMEM),
            pl.BlockSpec((1, 1, _NP), lambda i: (i, 0, 0)),
            pl.BlockSpec((1, 1, _NP), lambda i: (i, 0, 0)),
            pl.BlockSpec((1, 4, _NP), lambda i: (i, 0, 0)),
            pl.BlockSpec((1, 4, _NP), lambda i: (0, 0, 0)),
        ],
        out_specs=[pl.BlockSpec((1, 6, _NP), lambda i: (i, 0, 0))],
        out_shape=[jax.ShapeDtypeStruct((B, 6, _NP), jnp.float32)],
        compiler_params=pltpu.CompilerParams(dimension_semantics=("parallel",)),
    )(image_sizes, thr, s_rowp, l_rowp, reg_rowp, anch_rowp)[0]

    thrvec = jnp.full((_L,), jnp.asarray(nms_thresh, jnp.float32))
    mesh = plsc.VectorSubcoreMesh(core_axis_name="c", subcore_axis_name="s")
    cp = pltpu.CompilerParams()
    if "needs_layout_passes" in pltpu.CompilerParams.__dataclass_fields__:
        cp = dataclasses.replace(cp, needs_layout_passes=False)
    sc_nms = functools.partial(
        pl.kernel, mesh=mesh, compiler_params=cp,
        out_type=jax.ShapeDtypeStruct((B, _NP), jnp.int32),
        scratch_types=[pltpu.VMEM((_NP,), jnp.float32)] * 6
        + [pltpu.VMEM((_CAP,), jnp.float32)] * 5
        + [pltpu.VMEM((_CAP,), jnp.int32),
           pltpu.VMEM((_CAP,), jnp.float32),
           pltpu.VMEM((_NP,), jnp.int32),
           pltpu.VMEM((_NP,), jnp.int32),
           pltpu.VMEM((_L,), jnp.float32),
           pltpu.VMEM_SHARED((16, _NP), jnp.int32)],
    )(_sc_nms_body)
    keep_sc = sc_nms(data, thrvec)

    keep_rowp = keep_sc.reshape(B, 1, _NP)
    boxes4, sco, lab, keepi = pl.pallas_call(
        _final_kernel,
        grid=(B,),
        in_specs=[
            pl.BlockSpec(memory_space=pltpu.SMEM),
            pl.BlockSpec(memory_space=pltpu.SMEM),
            pl.BlockSpec((1, 1, _NP), lambda i: (i, 0, 0)),
            pl.BlockSpec((1, 1, _NP), lambda i: (i, 0, 0)),
            pl.BlockSpec((1, 4, _NP), lambda i: (i, 0, 0)),
            pl.BlockSpec((1, 4, _NP), lambda i: (0, 0, 0)),
            pl.BlockSpec((1, 1, _NP), lambda i: (i, 0, 0)),
        ],
        out_specs=[pl.BlockSpec((1, 4, _NP), lambda i: (i, 0, 0)),
                   pl.BlockSpec((1, 1, _NP), lambda i: (i, 0, 0)),
                   pl.BlockSpec((1, 1, _NP), lambda i: (i, 0, 0)),
                   pl.BlockSpec((1, 1, _NP), lambda i: (i, 0, 0))],
        out_shape=[jax.ShapeDtypeStruct((B, 4, _NP), jnp.float32),
                   jax.ShapeDtypeStruct((B, 1, _NP), jnp.float32),
                   jax.ShapeDtypeStruct((B, 1, _NP), jnp.int32),
                   jax.ShapeDtypeStruct((B, 1, _NP), jnp.int32)],
        compiler_params=pltpu.CompilerParams(dimension_semantics=("parallel",)),
    )(image_sizes, image_sizes_ori, s_rowp, l_rowp, reg_rowp, anch_rowp,
      keep_rowp)

    boxes = jnp.transpose(boxes4, (0, 2, 1))[:, :n, :]
    scores = sco[:, 0, :n]
    labels = lab[:, 0, :n]
    keep = keepi[:, 0, :n].astype(bool)
    return boxes, scores, labels, keep
